# Initial kernel scaffold; baseline (speedup 1.0000x reference)
import jax
import jax.numpy as jnp
from jax.experimental import pallas as pl


def kernel(x, edge_index, edge_attr, Wn1, bn1, Ws1, bs1, g1, be1, Wn2, bn2, Ws2, bs2, g2, be2, Wn3, bn3, Ws3, bs3, g3, be3):
    raise NotImplementedError("write your pallas kernel here")


# speedup vs baseline: 5.9390x; 5.9390x over previous
import functools

import jax
import jax.numpy as jnp
from jax import lax
from jax.experimental import pallas as pl
from jax.experimental.pallas import tpu as pltpu
from jax.experimental.pallas import tpu_sc as plsc

N = 10000
E = 320000
D = 128
DE = 16
EPS = 1e-5

NC = 2
NS = 16
NW = NC * NS
CH = 128
KCH = 80
EW = CH * KCH
EP = EW * NW
PAD = EP - E
NP = 10240
RT = NP // NS
RB = RT // CH


def _sc_seg_body(h_hbm, src_hbm, dst_hbm, zeros_hbm, out_hbm,
                 src_v, dst_v, rows_v, acc, sem):
    c = lax.axis_index("c")
    s = lax.axis_index("s")
    wid = c * NS + s
    pltpu.sync_copy(zeros_hbm, rows_v)
    for b in range(RB):
        pltpu.sync_copy(rows_v, acc.at[pl.ds(s * RT + b * CH, CH)])
    pltpu.sync_copy(src_hbm.at[pl.ds(wid * KCH, KCH)], src_v)
    pltpu.sync_copy(dst_hbm.at[pl.ds(wid * KCH, KCH)], dst_v)
    plsc.subcore_barrier()

    def body(j, carry):
        pltpu.async_copy(h_hbm.at[src_v.at[j]], rows_v, sem).wait()
        pltpu.sync_copy(rows_v, acc.at[dst_v.at[j]], add=True)
        return carry

    lax.fori_loop(0, KCH, body, 0)
    plsc.subcore_barrier()
    for b in range(RB):
        r0 = s * RT + b * CH
        pltpu.sync_copy(acc.at[pl.ds(r0, CH)], rows_v)
        pltpu.sync_copy(rows_v, out_hbm.at[pl.ds(c * NP + r0, CH)])


@functools.cache
def _make_sc_seg():
    return functools.partial(
        pl.kernel,
        mesh=plsc.VectorSubcoreMesh(core_axis_name="c", subcore_axis_name="s"),
        out_type=jax.ShapeDtypeStruct((NC * NP, D), jnp.float32),
        scratch_types=[
            pltpu.VMEM((KCH, CH), jnp.int32),
            pltpu.VMEM((KCH, CH), jnp.int32),
            pltpu.VMEM((CH, D), jnp.float32),
            pltpu.VMEM_SHARED((NP, D), jnp.float32),
            pltpu.SemaphoreType.DMA,
        ],
    )(_sc_seg_body)


def _dense_body(final_relu, h, p0, p1, q0, q1,
                wx, we, ws, bn, bs, g, be, out):
    ae = q0[:, :DE] + q1[:, :DE]
    cnt = q0[:, DE:DE + 1] + q1[:, DE:DE + 1]
    lin = (jnp.dot(p0[:] + p1[:], wx[:], preferred_element_type=jnp.float32)
           + jnp.dot(ae, we[:], preferred_element_type=jnp.float32)
           + cnt * bn[:])
    agg = lin / jnp.maximum(cnt, 1.0)
    t = jnp.maximum(
        jnp.dot(h[:], ws[:], preferred_element_type=jnp.float32) + bs[:] + agg,
        0.0)
    mu = jnp.mean(t, axis=0, keepdims=True)
    dlt = t - mu
    var = jnp.mean(dlt * dlt, axis=0, keepdims=True)
    y = dlt * lax.rsqrt(var + EPS) * g[:] + be[:]
    out[:] = jnp.maximum(y, 0.0) if final_relu else y


def _dense(final_relu, *args):
    return pl.pallas_call(
        functools.partial(_dense_body, final_relu),
        out_shape=jax.ShapeDtypeStruct((N, D), jnp.float32),
    )(*args)


def kernel(x, edge_index, edge_attr, Wn1, bn1, Ws1, bs1, g1, be1,
           Wn2, bn2, Ws2, bs2, g2, be2, Wn3, bn3, Ws3, bs3, g3, be3):
    src = edge_index[0]
    dst = edge_index[1]
    pad_ids = jnp.arange(PAD, dtype=jnp.int32)
    srcp = jnp.concatenate([src, pad_ids % N]).reshape(EP // CH, CH)
    dstp = jnp.concatenate(
        [dst, N + pad_ids % (NP - N)]).reshape(EP // CH, CH)
    zeros_h = jnp.zeros((CH, D), jnp.float32)

    ea128 = jnp.concatenate(
        [edge_attr, jnp.ones((E, 1), jnp.float32),
         jnp.zeros((E, D - DE - 1), jnp.float32)], axis=1)
    ea128 = jnp.concatenate([ea128, jnp.zeros((PAD, D), jnp.float32)], axis=0)
    eidx = jnp.arange(EP, dtype=jnp.int32).reshape(EP // CH, CH)

    seg = _make_sc_seg()
    q = seg(ea128, eidx, dstp, zeros_h)
    q0, q1 = q[:N], q[NP:NP + N]

    h = x
    layers = (
        (Wn1, bn1, Ws1, bs1, g1, be1, True),
        (Wn2, bn2, Ws2, bs2, g2, be2, True),
        (Wn3, bn3, Ws3, bs3, g3, be3, False),
    )
    for Wn, bn, Ws, bs, g, be, fr in layers:
        p = seg(h, srcp, dstp, zeros_h)
        h = _dense(fr, h, p[:N], p[NP:NP + N], q0, q1,
                   Wn[:, :D].T, Wn[:, D:].T, Ws.T,
                   bn[None], bs[None], g[None], be[None])
    return h
